# Initial kernel scaffold; baseline (speedup 1.0000x reference)
import jax
import jax.numpy as jnp
from jax.experimental import pallas as pl


def kernel(users, items, xij, edge_index, edge_vals, emb_user, emb_item, W_user, W_item, xij_item1, xij_item0):
    raise NotImplementedError("write your pallas kernel here")


# speedup vs baseline: 3.0973x; 3.0973x over previous
import functools

import jax
import jax.numpy as jnp
from jax import lax
from jax.experimental import pallas as pl
from jax.experimental.pallas import tpu as pltpu
from jax.experimental.pallas import tpu_sc as plsc

NUM_USERS = 5000
NUM_ITEMS = 5000
N = NUM_USERS + NUM_ITEMS
D = 128
E = 320000
B = 16384
NC = 2
NS = 16
NW = NC * NS
EPW = E // NW
ECHUNK = 80
ENCHUNKS = EPW // ECHUNK
NPAD = 10240
ROWS_PER_TILE = NPAD // NS
ZCHUNK = 128


def _propagate_layer(emb, src, dst, vals):
    mesh = plsc.VectorSubcoreMesh(core_axis_name="c", subcore_axis_name="s")

    @functools.partial(
        pl.kernel,
        out_type=jax.ShapeDtypeStruct((2 * NPAD, D), jnp.float32),
        mesh=mesh,
        scratch_types=[
            pltpu.VMEM((ECHUNK,), jnp.int32),
            pltpu.VMEM((ECHUNK,), jnp.int32),
            pltpu.VMEM((ECHUNK,), jnp.float32),
            pltpu.VMEM((ECHUNK, D), jnp.float32),
            pltpu.VMEM((ZCHUNK, D), jnp.float32),
            pltpu.VMEM_SHARED((NPAD, D), jnp.float32),
            pltpu.SemaphoreType.DMA,
        ],
        compiler_params=pltpu.CompilerParams(needs_layout_passes=False),
    )
    def k(emb_hbm, src_hbm, dst_hbm, vals_hbm, part_hbm,
          src_v, dst_v, vals_v, rows_v, stage_v, acc_sh, sem):
        c = lax.axis_index("c")
        s = lax.axis_index("s")
        wid = c * NS + s

        def _zrow(r, carry):
            for j in range(D // 16):
                stage_v[r, pl.ds(j * 16, 16)] = jnp.zeros((16,), jnp.float32)
            return carry
        lax.fori_loop(0, ZCHUNK, _zrow, 0)
        row0 = s * ROWS_PER_TILE
        for kk in range(ROWS_PER_TILE // ZCHUNK):
            pltpu.sync_copy(stage_v, acc_sh.at[pl.ds(row0 + kk * ZCHUNK, ZCHUNK)])
        plsc.subcore_barrier()

        ebase = wid * EPW

        def _edge_chunk(i, carry):
            off = ebase + i * ECHUNK
            pltpu.sync_copy(src_hbm.at[pl.ds(off, ECHUNK)], src_v)
            pltpu.sync_copy(dst_hbm.at[pl.ds(off, ECHUNK)], dst_v)
            pltpu.sync_copy(vals_hbm.at[pl.ds(off, ECHUNK)], vals_v)
            pltpu.async_copy(emb_hbm.at[src_v], rows_v, sem).wait()

            def _scale(e, c2):
                sp = plsc.load_gather(vals_v, [jnp.full((16,), e, jnp.int32)])
                for j in range(D // 16):
                    sl = pl.ds(j * 16, 16)
                    rows_v[e, sl] = rows_v[e, sl] * sp
                return c2
            lax.fori_loop(0, ECHUNK, _scale, 0)
            pltpu.sync_copy(rows_v, acc_sh.at[dst_v], add=True)
            return carry
        lax.fori_loop(0, ENCHUNKS, _edge_chunk, 0)
        plsc.subcore_barrier()

        out0 = c * NPAD + row0
        for kk in range(ROWS_PER_TILE // ZCHUNK):
            pltpu.sync_copy(acc_sh.at[pl.ds(row0 + kk * ZCHUNK, ZCHUNK)], stage_v)
            pltpu.sync_copy(stage_v, part_hbm.at[pl.ds(out0 + kk * ZCHUNK, ZCHUNK)])

    return k(emb, src, dst, vals)


def _combine(part, acc, last):
    BR = 80
    bs0 = pl.BlockSpec((BR, D), lambda i: (i, 0))
    bs1 = pl.BlockSpec((BR, D), lambda i: (i + NPAD // BR, 0))

    if last:
        def body(p0_ref, p1_ref, acc_ref, light_ref):
            e = p0_ref[...] + p1_ref[...]
            light_ref[...] = (acc_ref[...] + e) * 0.25
        out_shape = jax.ShapeDtypeStruct((N, D), jnp.float32)
        out_specs = bs0
    else:
        def body(p0_ref, p1_ref, acc_ref, emb_ref, accout_ref):
            e = p0_ref[...] + p1_ref[...]
            emb_ref[...] = e
            accout_ref[...] = acc_ref[...] + e
        out_shape = (jax.ShapeDtypeStruct((N, D), jnp.float32),
                     jax.ShapeDtypeStruct((N, D), jnp.float32))
        out_specs = (bs0, bs0)

    return pl.pallas_call(
        body,
        grid=(N // BR,),
        in_specs=[bs0, bs1, bs0],
        out_specs=out_specs,
        out_shape=out_shape,
    )(part, part, acc)


def _batch_gather(light, users, items, xij, x1, x0):
    mesh = plsc.VectorSubcoreMesh(core_axis_name="c", subcore_axis_name="s")
    BPW = B // NW
    CH = 128

    @functools.partial(
        pl.kernel,
        out_type=(jax.ShapeDtypeStruct((B, D), jnp.float32),
                  jax.ShapeDtypeStruct((B, D), jnp.float32),
                  jax.ShapeDtypeStruct((B,), jnp.float32)),
        mesh=mesh,
        scratch_types=[
            pltpu.VMEM((CH,), jnp.int32),
            pltpu.VMEM((CH,), jnp.int32),
            pltpu.VMEM((CH,), jnp.int32),
            pltpu.VMEM((CH,), jnp.float32),
            pltpu.VMEM((CH, D), jnp.float32),
            pltpu.VMEM((NUM_ITEMS,), jnp.float32),
            pltpu.VMEM((NUM_ITEMS,), jnp.float32),
            pltpu.SemaphoreType.DMA,
        ],
        compiler_params=pltpu.CompilerParams(needs_layout_passes=False),
    )
    def k(light_hbm, users_hbm, items_hbm, xij_hbm, x1_hbm, x0_hbm,
          urows_hbm, irows_hbm, xsel_hbm,
          uidx_v, iidx_v, xv_v, xsel_v, rows_v, x1_v, x0_v, sem):
        c = lax.axis_index("c")
        s = lax.axis_index("s")
        wid = c * NS + s
        pltpu.sync_copy(x1_hbm, x1_v)
        pltpu.sync_copy(x0_hbm, x0_v)
        base = wid * BPW
        for kk in range(BPW // CH):
            off = base + kk * CH
            pltpu.sync_copy(users_hbm.at[pl.ds(off, CH)], uidx_v)
            pltpu.async_copy(light_hbm.at[uidx_v], rows_v, sem).wait()
            pltpu.sync_copy(rows_v, urows_hbm.at[pl.ds(off, CH)])

            pltpu.sync_copy(items_hbm.at[pl.ds(off, CH)], iidx_v)
            pltpu.sync_copy(xij_hbm.at[pl.ds(off, CH)], xv_v)
            for g in range(CH // 16):
                sl = pl.ds(g * 16, 16)
                idx16 = iidx_v[sl]
                v1 = plsc.load_gather(x1_v, [idx16])
                v0 = plsc.load_gather(x0_v, [idx16])
                xsel_v[sl] = jnp.where(xv_v[sl] != 0, v1, v0)
                iidx_v[sl] = idx16 + NUM_USERS
            pltpu.sync_copy(xsel_v, xsel_hbm.at[pl.ds(off, CH)])
            pltpu.async_copy(light_hbm.at[iidx_v], rows_v, sem).wait()
            pltpu.sync_copy(rows_v, irows_hbm.at[pl.ds(off, CH)])

    return k(light, users, items, xij, x1, x0)


def _score(u_rows, i_rows, xsel2d, wu, wi):
    BT = 2048

    def body(u_ref, i_ref, x_ref, wu_ref, wi_ref, out_ref):
        lu = lax.dot_general(u_ref[...], wu_ref[...], (((1,), (1,)), ((), ())),
                             preferred_element_type=jnp.float32)
        m = jnp.max(lu, axis=1, keepdims=True)
        ex = jnp.exp(lu - m)
        p = ex / jnp.sum(ex, axis=1, keepdims=True)
        li = lax.dot_general(i_ref[...], wi_ref[...], (((1,), (1,)), ((), ())),
                             preferred_element_type=jnp.float32)
        sg = 1.0 / (1.0 + jnp.exp(-li))
        prod = (p * sg).reshape(BT // 128, 128, 128)
        g = 0.5 * jnp.sum(prod, axis=2)
        out_ref[...] = g + 0.5 / (1.0 + jnp.exp(-x_ref[...]))

    return pl.pallas_call(
        body,
        grid=(B // BT,),
        in_specs=[pl.BlockSpec((BT, D), lambda i: (i, 0)),
                  pl.BlockSpec((BT, D), lambda i: (i, 0)),
                  pl.BlockSpec((BT // 128, 128), lambda i: (i, 0)),
                  pl.BlockSpec((D, D), lambda i: (0, 0)),
                  pl.BlockSpec((D, D), lambda i: (0, 0))],
        out_specs=pl.BlockSpec((BT // 128, 128), lambda i: (i, 0)),
        out_shape=jax.ShapeDtypeStruct((B // 128, 128), jnp.float32),
    )(u_rows, i_rows, xsel2d, wu, wi)


def kernel(users, items, xij, edge_index, edge_vals, emb_user, emb_item,
           W_user, W_item, xij_item1, xij_item0):
    all_emb = jnp.concatenate([emb_user, emb_item], axis=0)
    src = edge_index[0]
    dst = edge_index[1]
    emb = all_emb
    acc = all_emb
    light = None
    for layer in range(3):
        part = _propagate_layer(emb, src, dst, edge_vals)
        if layer < 2:
            emb, acc = _combine(part, acc, last=False)
        else:
            light = _combine(part, acc, last=True)
    u_rows, i_rows, xsel = _batch_gather(
        light, users, items, xij,
        xij_item1.reshape(-1), xij_item0.reshape(-1))
    gamma2d = _score(u_rows, i_rows, xsel.reshape(B // 128, 128),
                     W_user, W_item)
    return gamma2d.reshape(B)


# speedup vs baseline: 5.4242x; 1.7513x over previous
import functools

import jax
import jax.numpy as jnp
from jax import lax
from jax.experimental import pallas as pl
from jax.experimental.pallas import tpu as pltpu
from jax.experimental.pallas import tpu_sc as plsc

NUM_USERS = 5000
NUM_ITEMS = 5000
N = NUM_USERS + NUM_ITEMS
D = 128
E = 320000
B = 16384
NC = 2
NS = 16
NW = NC * NS
EPW = E // NW
ECHUNK = 80
ENCHUNKS = EPW // ECHUNK
EPAIRS = ENCHUNKS // 2
NPAD = 10240
ROWS_PER_TILE = NPAD // NS
ZCHUNK = ECHUNK


def _propagate_layer(emb, src, dst, vals):
    mesh = plsc.VectorSubcoreMesh(core_axis_name="c", subcore_axis_name="s")

    @functools.partial(
        pl.kernel,
        out_type=jax.ShapeDtypeStruct((2 * NPAD, D), jnp.float32),
        mesh=mesh,
        scratch_types=[
            pltpu.VMEM((ECHUNK,), jnp.int32),
            pltpu.VMEM((ECHUNK,), jnp.int32),
            pltpu.VMEM((ECHUNK,), jnp.float32),
            pltpu.VMEM((ECHUNK,), jnp.int32),
            pltpu.VMEM((ECHUNK,), jnp.int32),
            pltpu.VMEM((ECHUNK,), jnp.float32),
            pltpu.VMEM((ECHUNK, D), jnp.float32),
            pltpu.VMEM((ECHUNK, D), jnp.float32),
            pltpu.VMEM_SHARED((NPAD, D), jnp.float32),
            pltpu.SemaphoreType.DMA,
            pltpu.SemaphoreType.DMA,
            pltpu.SemaphoreType.DMA,
            pltpu.SemaphoreType.DMA,
        ],
        compiler_params=pltpu.CompilerParams(needs_layout_passes=False),
    )
    def k(emb_hbm, src_hbm, dst_hbm, vals_hbm, part_hbm,
          sa_v, da_v, va_v, sb_v, db_v, vb_v, rows_a, rows_b, acc_sh,
          isem_a, isem_b, gsem_a, gsem_b):
        c = lax.axis_index("c")
        s = lax.axis_index("s")
        wid = c * NS + s
        ebase = wid * EPW

        def _idx_start(i, sv, dv, vv, sem):
            off = ebase + i * ECHUNK
            pltpu.async_copy(src_hbm.at[pl.ds(off, ECHUNK)], sv, sem)
            pltpu.async_copy(dst_hbm.at[pl.ds(off, ECHUNK)], dv, sem)
            pltpu.async_copy(vals_hbm.at[pl.ds(off, ECHUNK)], vv, sem)

        def _idx_wait(i, sv, dv, vv, sem):
            off = ebase + i * ECHUNK
            pltpu.make_async_copy(src_hbm.at[pl.ds(off, ECHUNK)], sv, sem).wait()
            pltpu.make_async_copy(dst_hbm.at[pl.ds(off, ECHUNK)], dv, sem).wait()
            pltpu.make_async_copy(vals_hbm.at[pl.ds(off, ECHUNK)], vv, sem).wait()

        def _gather(sv, buf, sem):
            pltpu.async_copy(emb_hbm.at[sv], buf, sem)

        def _gwait(sv, buf, sem):
            pltpu.make_async_copy(emb_hbm.at[sv], buf, sem).wait()

        def _scale_scatter(buf, vv, dv):
            def _e(e, c2):
                sp = plsc.load_gather(vv, [jnp.full((16,), e, jnp.int32)])
                for j in range(D // 16):
                    sl = pl.ds(j * 16, 16)
                    buf[e, sl] = buf[e, sl] * sp
                return c2
            lax.fori_loop(0, ECHUNK, _e, 0, unroll=2)
            pltpu.sync_copy(buf, acc_sh.at[dv], add=True)

        def _zrow(r, carry):
            for j in range(D // 16):
                rows_a[r, pl.ds(j * 16, 16)] = jnp.zeros((16,), jnp.float32)
            return carry
        lax.fori_loop(0, ZCHUNK, _zrow, 0)
        row0 = s * ROWS_PER_TILE
        for kk in range(ROWS_PER_TILE // ZCHUNK):
            pltpu.sync_copy(rows_a, acc_sh.at[pl.ds(row0 + kk * ZCHUNK, ZCHUNK)])

        pltpu.sync_copy(src_hbm.at[pl.ds(ebase, ECHUNK)], sa_v)
        pltpu.sync_copy(dst_hbm.at[pl.ds(ebase, ECHUNK)], da_v)
        pltpu.sync_copy(vals_hbm.at[pl.ds(ebase, ECHUNK)], va_v)
        _gather(sa_v, rows_a, gsem_a)
        _idx_start(1, sb_v, db_v, vb_v, isem_b)
        plsc.subcore_barrier()

        def _pair(p, carry):
            i0 = 2 * p
            _idx_wait(i0 + 1, sb_v, db_v, vb_v, isem_b)
            _gather(sb_v, rows_b, gsem_b)
            _gwait(sa_v, rows_a, gsem_a)
            _scale_scatter(rows_a, va_v, da_v)
            _idx_start(i0 + 2, sa_v, da_v, va_v, isem_a)
            _gwait(sb_v, rows_b, gsem_b)
            _idx_wait(i0 + 2, sa_v, da_v, va_v, isem_a)
            _gather(sa_v, rows_a, gsem_a)
            _scale_scatter(rows_b, vb_v, db_v)

            @pl.when(p < EPAIRS - 1)
            def _():
                _idx_start(i0 + 3, sb_v, db_v, vb_v, isem_b)
            return carry
        lax.fori_loop(0, EPAIRS, _pair, 0)
        _gwait(sa_v, rows_a, gsem_a)
        _scale_scatter(rows_a, va_v, da_v)
        plsc.subcore_barrier()

        out0 = c * NPAD + row0
        for kk in range(ROWS_PER_TILE // ZCHUNK):
            pltpu.sync_copy(acc_sh.at[pl.ds(row0 + kk * ZCHUNK, ZCHUNK)], rows_a)
            pltpu.sync_copy(rows_a, part_hbm.at[pl.ds(out0 + kk * ZCHUNK, ZCHUNK)])

    return k(emb, src, dst, vals)


def _combine(part, acc, last):
    BR = 80
    bs0 = pl.BlockSpec((BR, D), lambda i: (i, 0))
    bs1 = pl.BlockSpec((BR, D), lambda i: (i + NPAD // BR, 0))

    if last:
        def body(p0_ref, p1_ref, acc_ref, light_ref):
            e = p0_ref[...] + p1_ref[...]
            light_ref[...] = (acc_ref[...] + e) * 0.25
        out_shape = jax.ShapeDtypeStruct((N, D), jnp.float32)
        out_specs = bs0
    else:
        def body(p0_ref, p1_ref, acc_ref, emb_ref, accout_ref):
            e = p0_ref[...] + p1_ref[...]
            emb_ref[...] = e
            accout_ref[...] = acc_ref[...] + e
        out_shape = (jax.ShapeDtypeStruct((N, D), jnp.float32),
                     jax.ShapeDtypeStruct((N, D), jnp.float32))
        out_specs = (bs0, bs0)

    return pl.pallas_call(
        body,
        grid=(N // BR,),
        in_specs=[bs0, bs1, bs0],
        out_specs=out_specs,
        out_shape=out_shape,
    )(part, part, acc)


def _batch_gather(light, users, items, xij, x1, x0):
    mesh = plsc.VectorSubcoreMesh(core_axis_name="c", subcore_axis_name="s")
    BPW = B // NW
    CH = 128

    @functools.partial(
        pl.kernel,
        out_type=(jax.ShapeDtypeStruct((B, D), jnp.float32),
                  jax.ShapeDtypeStruct((B, D), jnp.float32),
                  jax.ShapeDtypeStruct((B,), jnp.float32)),
        mesh=mesh,
        scratch_types=[
            pltpu.VMEM((CH,), jnp.int32),
            pltpu.VMEM((CH,), jnp.int32),
            pltpu.VMEM((CH,), jnp.int32),
            pltpu.VMEM((CH,), jnp.float32),
            pltpu.VMEM((CH, D), jnp.float32),
            pltpu.VMEM((NUM_ITEMS,), jnp.float32),
            pltpu.VMEM((NUM_ITEMS,), jnp.float32),
            pltpu.SemaphoreType.DMA,
        ],
        compiler_params=pltpu.CompilerParams(needs_layout_passes=False),
    )
    def k(light_hbm, users_hbm, items_hbm, xij_hbm, x1_hbm, x0_hbm,
          urows_hbm, irows_hbm, xsel_hbm,
          uidx_v, iidx_v, xv_v, xsel_v, rows_v, x1_v, x0_v, sem):
        c = lax.axis_index("c")
        s = lax.axis_index("s")
        wid = c * NS + s
        pltpu.sync_copy(x1_hbm, x1_v)
        pltpu.sync_copy(x0_hbm, x0_v)
        base = wid * BPW
        for kk in range(BPW // CH):
            off = base + kk * CH
            pltpu.sync_copy(users_hbm.at[pl.ds(off, CH)], uidx_v)
            pltpu.async_copy(light_hbm.at[uidx_v], rows_v, sem).wait()
            pltpu.sync_copy(rows_v, urows_hbm.at[pl.ds(off, CH)])

            pltpu.sync_copy(items_hbm.at[pl.ds(off, CH)], iidx_v)
            pltpu.sync_copy(xij_hbm.at[pl.ds(off, CH)], xv_v)
            for g in range(CH // 16):
                sl = pl.ds(g * 16, 16)
                idx16 = iidx_v[sl]
                v1 = plsc.load_gather(x1_v, [idx16])
                v0 = plsc.load_gather(x0_v, [idx16])
                xsel_v[sl] = jnp.where(xv_v[sl] != 0, v1, v0)
                iidx_v[sl] = idx16 + NUM_USERS
            pltpu.sync_copy(xsel_v, xsel_hbm.at[pl.ds(off, CH)])
            pltpu.async_copy(light_hbm.at[iidx_v], rows_v, sem).wait()
            pltpu.sync_copy(rows_v, irows_hbm.at[pl.ds(off, CH)])

    return k(light, users, items, xij, x1, x0)


def _score(u_rows, i_rows, xsel2d, wu, wi):
    BT = 2048

    def body(u_ref, i_ref, x_ref, wu_ref, wi_ref, out_ref):
        lu = lax.dot_general(u_ref[...], wu_ref[...], (((1,), (1,)), ((), ())),
                             preferred_element_type=jnp.float32)
        m = jnp.max(lu, axis=1, keepdims=True)
        ex = jnp.exp(lu - m)
        p = ex / jnp.sum(ex, axis=1, keepdims=True)
        li = lax.dot_general(i_ref[...], wi_ref[...], (((1,), (1,)), ((), ())),
                             preferred_element_type=jnp.float32)
        sg = 1.0 / (1.0 + jnp.exp(-li))
        prod = (p * sg).reshape(BT // 128, 128, 128)
        g = 0.5 * jnp.sum(prod, axis=2)
        out_ref[...] = g + 0.5 / (1.0 + jnp.exp(-x_ref[...]))

    return pl.pallas_call(
        body,
        grid=(B // BT,),
        in_specs=[pl.BlockSpec((BT, D), lambda i: (i, 0)),
                  pl.BlockSpec((BT, D), lambda i: (i, 0)),
                  pl.BlockSpec((BT // 128, 128), lambda i: (i, 0)),
                  pl.BlockSpec((D, D), lambda i: (0, 0)),
                  pl.BlockSpec((D, D), lambda i: (0, 0))],
        out_specs=pl.BlockSpec((BT // 128, 128), lambda i: (i, 0)),
        out_shape=jax.ShapeDtypeStruct((B // 128, 128), jnp.float32),
    )(u_rows, i_rows, xsel2d, wu, wi)


def kernel(users, items, xij, edge_index, edge_vals, emb_user, emb_item,
           W_user, W_item, xij_item1, xij_item0):
    all_emb = jnp.concatenate([emb_user, emb_item], axis=0)
    src = edge_index[0]
    dst = edge_index[1]
    emb = all_emb
    acc = all_emb
    light = None
    for layer in range(3):
        part = _propagate_layer(emb, src, dst, edge_vals)
        if layer < 2:
            emb, acc = _combine(part, acc, last=False)
        else:
            light = _combine(part, acc, last=True)
    u_rows, i_rows, xsel = _batch_gather(
        light, users, items, xij,
        xij_item1.reshape(-1), xij_item0.reshape(-1))
    gamma2d = _score(u_rows, i_rows, xsel.reshape(B // 128, 128),
                     W_user, W_item)
    return gamma2d.reshape(B)


# speedup vs baseline: 6.1171x; 1.1277x over previous
import functools

import jax
import jax.numpy as jnp
from jax import lax
from jax.experimental import pallas as pl
from jax.experimental.pallas import tpu as pltpu
from jax.experimental.pallas import tpu_sc as plsc

NUM_USERS = 5000
NUM_ITEMS = 5000
N = NUM_USERS + NUM_ITEMS
D = 128
E = 320000
B = 16384
NC = 2
NS = 16
NW = NC * NS
EPW = E // NW
ECHUNK = 128
ENCHUNKS = EPW // ECHUNK
ETAIL = EPW - ENCHUNKS * ECHUNK
EPAIRS = ENCHUNKS // 2
NPAD = 10240
ROWS_PER_TILE = NPAD // NS
ZCHUNK = ECHUNK


def _propagate_layer(emb, src, dst, vals):
    mesh = plsc.VectorSubcoreMesh(core_axis_name="c", subcore_axis_name="s")

    @functools.partial(
        pl.kernel,
        out_type=jax.ShapeDtypeStruct((2 * NPAD, D), jnp.float32),
        mesh=mesh,
        scratch_types=[
            pltpu.VMEM((ECHUNK,), jnp.int32),
            pltpu.VMEM((ECHUNK,), jnp.int32),
            pltpu.VMEM((ECHUNK,), jnp.float32),
            pltpu.VMEM((ECHUNK,), jnp.int32),
            pltpu.VMEM((ECHUNK,), jnp.int32),
            pltpu.VMEM((ECHUNK,), jnp.float32),
            pltpu.VMEM((ETAIL,), jnp.int32),
            pltpu.VMEM((ETAIL,), jnp.int32),
            pltpu.VMEM((ETAIL,), jnp.float32),
            pltpu.VMEM((ECHUNK, D), jnp.float32),
            pltpu.VMEM((ECHUNK, D), jnp.float32),
            pltpu.VMEM((ETAIL, D), jnp.float32),
            pltpu.VMEM_SHARED((NPAD, D), jnp.float32),
            pltpu.SemaphoreType.DMA,
            pltpu.SemaphoreType.DMA,
            pltpu.SemaphoreType.DMA,
            pltpu.SemaphoreType.DMA,
            pltpu.SemaphoreType.DMA,
            pltpu.SemaphoreType.DMA,
        ],
        compiler_params=pltpu.CompilerParams(needs_layout_passes=False),
    )
    def k(emb_hbm, src_hbm, dst_hbm, vals_hbm, part_hbm,
          sa_v, da_v, va_v, sb_v, db_v, vb_v, st_v, dt_v, vt_v,
          rows_a, rows_b, rows_t, acc_sh,
          isem_a, isem_b, isem_t, gsem_a, gsem_b, ssem):
        c = lax.axis_index("c")
        s = lax.axis_index("s")
        wid = c * NS + s
        ebase = wid * EPW

        def _idx_start(i, sv, dv, vv, sem):
            off = ebase + i * ECHUNK
            pltpu.async_copy(src_hbm.at[pl.ds(off, ECHUNK)], sv, sem)
            pltpu.async_copy(dst_hbm.at[pl.ds(off, ECHUNK)], dv, sem)
            pltpu.async_copy(vals_hbm.at[pl.ds(off, ECHUNK)], vv, sem)

        def _idx_wait(i, sv, dv, vv, sem):
            off = ebase + i * ECHUNK
            pltpu.make_async_copy(src_hbm.at[pl.ds(off, ECHUNK)], sv, sem).wait()
            pltpu.make_async_copy(dst_hbm.at[pl.ds(off, ECHUNK)], dv, sem).wait()
            pltpu.make_async_copy(vals_hbm.at[pl.ds(off, ECHUNK)], vv, sem).wait()

        def _gather(sv, buf, sem):
            pltpu.async_copy(emb_hbm.at[sv], buf, sem)

        def _gwait(sv, buf, sem):
            pltpu.make_async_copy(emb_hbm.at[sv], buf, sem).wait()

        def _scale(buf, vv, n):
            def _e(e, c2):
                sp = plsc.load_gather(vv, [jnp.full((16,), e, jnp.int32)])
                for j in range(D // 16):
                    sl = pl.ds(j * 16, 16)
                    buf[e, sl] = buf[e, sl] * sp
                return c2
            lax.fori_loop(0, n, _e, 0, unroll=2)

        def _zrow(r, carry):
            for j in range(D // 16):
                rows_a[r, pl.ds(j * 16, 16)] = jnp.zeros((16,), jnp.float32)
            return carry
        lax.fori_loop(0, ZCHUNK, _zrow, 0)
        row0 = s * ROWS_PER_TILE
        for kk in range(ROWS_PER_TILE // ZCHUNK):
            pltpu.sync_copy(rows_a, acc_sh.at[pl.ds(row0 + kk * ZCHUNK, ZCHUNK)])

        pltpu.sync_copy(src_hbm.at[pl.ds(ebase, ECHUNK)], sa_v)
        pltpu.sync_copy(dst_hbm.at[pl.ds(ebase, ECHUNK)], da_v)
        pltpu.sync_copy(vals_hbm.at[pl.ds(ebase, ECHUNK)], va_v)
        _gather(sa_v, rows_a, gsem_a)
        _idx_start(1, sb_v, db_v, vb_v, isem_b)
        toff = ebase + ENCHUNKS * ECHUNK
        pltpu.async_copy(src_hbm.at[pl.ds(toff, ETAIL)], st_v, isem_t)
        pltpu.async_copy(dst_hbm.at[pl.ds(toff, ETAIL)], dt_v, isem_t)
        pltpu.async_copy(vals_hbm.at[pl.ds(toff, ETAIL)], vt_v, isem_t)
        plsc.subcore_barrier()

        def _pair(p, carry):
            i0 = 2 * p
            _idx_wait(i0 + 1, sb_v, db_v, vb_v, isem_b)
            _gather(sb_v, rows_b, gsem_b)
            _gwait(sa_v, rows_a, gsem_a)
            _scale(rows_a, va_v, ECHUNK)
            pltpu.async_copy(rows_a, acc_sh.at[da_v], ssem, add=True)
            _gwait(sb_v, rows_b, gsem_b)
            _scale(rows_b, vb_v, ECHUNK)
            pltpu.make_async_copy(rows_a, acc_sh.at[da_v], ssem).wait()

            @pl.when(p < EPAIRS - 1)
            def _():
                _idx_start(i0 + 2, sa_v, da_v, va_v, isem_a)
                _idx_wait(i0 + 2, sa_v, da_v, va_v, isem_a)
                _gather(sa_v, rows_a, gsem_a)
            pltpu.sync_copy(rows_b, acc_sh.at[db_v], add=True)

            @pl.when(p < EPAIRS - 1)
            def _():
                _idx_start(i0 + 3, sb_v, db_v, vb_v, isem_b)
            return carry
        lax.fori_loop(0, EPAIRS, _pair, 0)
        pltpu.make_async_copy(src_hbm.at[pl.ds(toff, ETAIL)], st_v, isem_t).wait()
        pltpu.make_async_copy(dst_hbm.at[pl.ds(toff, ETAIL)], dt_v, isem_t).wait()
        pltpu.make_async_copy(vals_hbm.at[pl.ds(toff, ETAIL)], vt_v, isem_t).wait()
        pltpu.async_copy(emb_hbm.at[st_v], rows_t, gsem_a)
        pltpu.make_async_copy(emb_hbm.at[st_v], rows_t, gsem_a).wait()
        _scale(rows_t, vt_v, ETAIL)
        pltpu.sync_copy(rows_t, acc_sh.at[dt_v], add=True)
        plsc.subcore_barrier()

        out0 = c * NPAD + row0
        for kk in range(ROWS_PER_TILE // ZCHUNK):
            pltpu.sync_copy(acc_sh.at[pl.ds(row0 + kk * ZCHUNK, ZCHUNK)], rows_a)
            pltpu.sync_copy(rows_a, part_hbm.at[pl.ds(out0 + kk * ZCHUNK, ZCHUNK)])

    return k(emb, src, dst, vals)


def _combine(part, acc, last):
    BR = 80
    bs0 = pl.BlockSpec((BR, D), lambda i: (i, 0))
    bs1 = pl.BlockSpec((BR, D), lambda i: (i + NPAD // BR, 0))

    if last:
        def body(p0_ref, p1_ref, acc_ref, light_ref):
            e = p0_ref[...] + p1_ref[...]
            light_ref[...] = (acc_ref[...] + e) * 0.25
        out_shape = jax.ShapeDtypeStruct((N, D), jnp.float32)
        out_specs = bs0
    else:
        def body(p0_ref, p1_ref, acc_ref, emb_ref, accout_ref):
            e = p0_ref[...] + p1_ref[...]
            emb_ref[...] = e
            accout_ref[...] = acc_ref[...] + e
        out_shape = (jax.ShapeDtypeStruct((N, D), jnp.float32),
                     jax.ShapeDtypeStruct((N, D), jnp.float32))
        out_specs = (bs0, bs0)

    return pl.pallas_call(
        body,
        grid=(N // BR,),
        in_specs=[bs0, bs1, bs0],
        out_specs=out_specs,
        out_shape=out_shape,
    )(part, part, acc)


def _batch_gather(light, users, items, xij, x1, x0):
    mesh = plsc.VectorSubcoreMesh(core_axis_name="c", subcore_axis_name="s")
    BPW = B // NW
    CH = 128

    @functools.partial(
        pl.kernel,
        out_type=(jax.ShapeDtypeStruct((B, D), jnp.float32),
                  jax.ShapeDtypeStruct((B, D), jnp.float32),
                  jax.ShapeDtypeStruct((B,), jnp.float32)),
        mesh=mesh,
        scratch_types=[
            pltpu.VMEM((CH,), jnp.int32),
            pltpu.VMEM((CH,), jnp.int32),
            pltpu.VMEM((CH,), jnp.int32),
            pltpu.VMEM((CH,), jnp.float32),
            pltpu.VMEM((CH, D), jnp.float32),
            pltpu.VMEM((NUM_ITEMS,), jnp.float32),
            pltpu.VMEM((NUM_ITEMS,), jnp.float32),
            pltpu.SemaphoreType.DMA,
        ],
        compiler_params=pltpu.CompilerParams(needs_layout_passes=False),
    )
    def k(light_hbm, users_hbm, items_hbm, xij_hbm, x1_hbm, x0_hbm,
          urows_hbm, irows_hbm, xsel_hbm,
          uidx_v, iidx_v, xv_v, xsel_v, rows_v, x1_v, x0_v, sem):
        c = lax.axis_index("c")
        s = lax.axis_index("s")
        wid = c * NS + s
        pltpu.sync_copy(x1_hbm, x1_v)
        pltpu.sync_copy(x0_hbm, x0_v)
        base = wid * BPW
        for kk in range(BPW // CH):
            off = base + kk * CH
            pltpu.sync_copy(users_hbm.at[pl.ds(off, CH)], uidx_v)
            pltpu.async_copy(light_hbm.at[uidx_v], rows_v, sem).wait()
            pltpu.sync_copy(rows_v, urows_hbm.at[pl.ds(off, CH)])

            pltpu.sync_copy(items_hbm.at[pl.ds(off, CH)], iidx_v)
            pltpu.sync_copy(xij_hbm.at[pl.ds(off, CH)], xv_v)
            for g in range(CH // 16):
                sl = pl.ds(g * 16, 16)
                idx16 = iidx_v[sl]
                v1 = plsc.load_gather(x1_v, [idx16])
                v0 = plsc.load_gather(x0_v, [idx16])
                xsel_v[sl] = jnp.where(xv_v[sl] != 0, v1, v0)
                iidx_v[sl] = idx16 + NUM_USERS
            pltpu.sync_copy(xsel_v, xsel_hbm.at[pl.ds(off, CH)])
            pltpu.async_copy(light_hbm.at[iidx_v], rows_v, sem).wait()
            pltpu.sync_copy(rows_v, irows_hbm.at[pl.ds(off, CH)])

    return k(light, users, items, xij, x1, x0)


def _score(u_rows, i_rows, xsel2d, wu, wi):
    BT = 2048

    def body(u_ref, i_ref, x_ref, wu_ref, wi_ref, out_ref):
        lu = lax.dot_general(u_ref[...], wu_ref[...], (((1,), (1,)), ((), ())),
                             preferred_element_type=jnp.float32)
        m = jnp.max(lu, axis=1, keepdims=True)
        ex = jnp.exp(lu - m)
        p = ex / jnp.sum(ex, axis=1, keepdims=True)
        li = lax.dot_general(i_ref[...], wi_ref[...], (((1,), (1,)), ((), ())),
                             preferred_element_type=jnp.float32)
        sg = 1.0 / (1.0 + jnp.exp(-li))
        prod = (p * sg).reshape(BT // 128, 128, 128)
        g = 0.5 * jnp.sum(prod, axis=2)
        out_ref[...] = g + 0.5 / (1.0 + jnp.exp(-x_ref[...]))

    return pl.pallas_call(
        body,
        grid=(B // BT,),
        in_specs=[pl.BlockSpec((BT, D), lambda i: (i, 0)),
                  pl.BlockSpec((BT, D), lambda i: (i, 0)),
                  pl.BlockSpec((BT // 128, 128), lambda i: (i, 0)),
                  pl.BlockSpec((D, D), lambda i: (0, 0)),
                  pl.BlockSpec((D, D), lambda i: (0, 0))],
        out_specs=pl.BlockSpec((BT // 128, 128), lambda i: (i, 0)),
        out_shape=jax.ShapeDtypeStruct((B // 128, 128), jnp.float32),
    )(u_rows, i_rows, xsel2d, wu, wi)


def kernel(users, items, xij, edge_index, edge_vals, emb_user, emb_item,
           W_user, W_item, xij_item1, xij_item0):
    all_emb = jnp.concatenate([emb_user, emb_item], axis=0)
    src = edge_index[0]
    dst = edge_index[1]
    emb = all_emb
    acc = all_emb
    light = None
    for layer in range(3):
        part = _propagate_layer(emb, src, dst, edge_vals)
        if layer < 2:
            emb, acc = _combine(part, acc, last=False)
        else:
            light = _combine(part, acc, last=True)
    u_rows, i_rows, xsel = _batch_gather(
        light, users, items, xij,
        xij_item1.reshape(-1), xij_item0.reshape(-1))
    gamma2d = _score(u_rows, i_rows, xsel.reshape(B // 128, 128),
                     W_user, W_item)
    return gamma2d.reshape(B)


# speedup vs baseline: 7.6720x; 1.2542x over previous
import functools

import jax
import jax.numpy as jnp
from jax import lax
from jax.experimental import pallas as pl
from jax.experimental.pallas import tpu as pltpu
from jax.experimental.pallas import tpu_sc as plsc

NUM_USERS = 5000
NUM_ITEMS = 5000
N = NUM_USERS + NUM_ITEMS
D = 128
E = 320000
B = 16384
NC = 2
NS = 16
NW = NC * NS
EPW = E // NW
ECHUNK = 128
ENCHUNKS = EPW // ECHUNK
ETAIL = EPW - ENCHUNKS * ECHUNK
EPAIRS = ENCHUNKS // 2
NPAD = 10240
ROWS_PER_TILE = NPAD // NS
ZCHUNK = ECHUNK


def _propagate_layer(emb, src, dst, vals):
    mesh = plsc.VectorSubcoreMesh(core_axis_name="c", subcore_axis_name="s")

    @functools.partial(
        pl.kernel,
        out_type=jax.ShapeDtypeStruct((2 * NPAD, D), jnp.float32),
        mesh=mesh,
        scratch_types=[
            pltpu.VMEM((ECHUNK,), jnp.int32),
            pltpu.VMEM((ECHUNK,), jnp.int32),
            pltpu.VMEM((ECHUNK,), jnp.float32),
            pltpu.VMEM((ECHUNK,), jnp.int32),
            pltpu.VMEM((ECHUNK,), jnp.int32),
            pltpu.VMEM((ECHUNK,), jnp.float32),
            pltpu.VMEM((ETAIL,), jnp.int32),
            pltpu.VMEM((ETAIL,), jnp.int32),
            pltpu.VMEM((ETAIL,), jnp.float32),
            pltpu.VMEM((ECHUNK, D), jnp.float32),
            pltpu.VMEM((ECHUNK, D), jnp.float32),
            pltpu.VMEM((ETAIL, D), jnp.float32),
            pltpu.VMEM_SHARED((NPAD, D), jnp.float32),
            pltpu.SemaphoreType.DMA,
            pltpu.SemaphoreType.DMA,
            pltpu.SemaphoreType.DMA,
            pltpu.SemaphoreType.DMA,
            pltpu.SemaphoreType.DMA,
            pltpu.SemaphoreType.DMA,
        ],
        compiler_params=pltpu.CompilerParams(needs_layout_passes=False),
    )
    def k(emb_hbm, src_hbm, dst_hbm, vals_hbm, part_hbm,
          sa_v, da_v, va_v, sb_v, db_v, vb_v, st_v, dt_v, vt_v,
          rows_a, rows_b, rows_t, acc_sh,
          isem_a, isem_b, isem_t, gsem_a, gsem_b, ssem):
        c = lax.axis_index("c")
        s = lax.axis_index("s")
        wid = c * NS + s
        ebase = wid * EPW

        def _idx_start(i, sv, dv, vv, sem):
            off = ebase + i * ECHUNK
            pltpu.async_copy(src_hbm.at[pl.ds(off, ECHUNK)], sv, sem)
            pltpu.async_copy(dst_hbm.at[pl.ds(off, ECHUNK)], dv, sem)
            pltpu.async_copy(vals_hbm.at[pl.ds(off, ECHUNK)], vv, sem)

        def _idx_wait(i, sv, dv, vv, sem):
            off = ebase + i * ECHUNK
            pltpu.make_async_copy(src_hbm.at[pl.ds(off, ECHUNK)], sv, sem).wait()
            pltpu.make_async_copy(dst_hbm.at[pl.ds(off, ECHUNK)], dv, sem).wait()
            pltpu.make_async_copy(vals_hbm.at[pl.ds(off, ECHUNK)], vv, sem).wait()

        def _gather(sv, buf, sem):
            pltpu.async_copy(emb_hbm.at[sv], buf, sem)

        def _gwait(sv, buf, sem):
            pltpu.make_async_copy(emb_hbm.at[sv], buf, sem).wait()

        def _scale(buf, vv, n):
            def _e(e, c2):
                sp = plsc.load_gather(vv, [jnp.full((16,), e, jnp.int32)])
                for j in range(D // 16):
                    sl = pl.ds(j * 16, 16)
                    buf[e, sl] = buf[e, sl] * sp
                return c2
            lax.fori_loop(0, n, _e, 0, unroll=2)

        def _zrow(r, carry):
            for j in range(D // 16):
                rows_a[r, pl.ds(j * 16, 16)] = jnp.zeros((16,), jnp.float32)
            return carry
        lax.fori_loop(0, ZCHUNK, _zrow, 0)
        row0 = s * ROWS_PER_TILE
        for kk in range(ROWS_PER_TILE // ZCHUNK):
            pltpu.sync_copy(rows_a, acc_sh.at[pl.ds(row0 + kk * ZCHUNK, ZCHUNK)])

        pltpu.sync_copy(src_hbm.at[pl.ds(ebase, ECHUNK)], sa_v)
        pltpu.sync_copy(dst_hbm.at[pl.ds(ebase, ECHUNK)], da_v)
        pltpu.sync_copy(vals_hbm.at[pl.ds(ebase, ECHUNK)], va_v)
        _gather(sa_v, rows_a, gsem_a)
        _idx_start(1, sb_v, db_v, vb_v, isem_b)
        toff = ebase + ENCHUNKS * ECHUNK
        pltpu.async_copy(src_hbm.at[pl.ds(toff, ETAIL)], st_v, isem_t)
        pltpu.async_copy(dst_hbm.at[pl.ds(toff, ETAIL)], dt_v, isem_t)
        pltpu.async_copy(vals_hbm.at[pl.ds(toff, ETAIL)], vt_v, isem_t)
        plsc.subcore_barrier()

        def _pair(p, carry):
            i0 = 2 * p
            _idx_wait(i0 + 1, sb_v, db_v, vb_v, isem_b)
            _gather(sb_v, rows_b, gsem_b)
            _gwait(sa_v, rows_a, gsem_a)
            _scale(rows_a, va_v, ECHUNK)
            pltpu.async_copy(rows_a, acc_sh.at[da_v], ssem, add=True)
            _gwait(sb_v, rows_b, gsem_b)
            _scale(rows_b, vb_v, ECHUNK)
            pltpu.make_async_copy(rows_a, acc_sh.at[da_v], ssem).wait()

            @pl.when(p < EPAIRS - 1)
            def _():
                _idx_start(i0 + 2, sa_v, da_v, va_v, isem_a)
                _idx_wait(i0 + 2, sa_v, da_v, va_v, isem_a)
                _gather(sa_v, rows_a, gsem_a)
            pltpu.sync_copy(rows_b, acc_sh.at[db_v], add=True)

            @pl.when(p < EPAIRS - 1)
            def _():
                _idx_start(i0 + 3, sb_v, db_v, vb_v, isem_b)
            return carry
        lax.fori_loop(0, EPAIRS, _pair, 0)
        pltpu.make_async_copy(src_hbm.at[pl.ds(toff, ETAIL)], st_v, isem_t).wait()
        pltpu.make_async_copy(dst_hbm.at[pl.ds(toff, ETAIL)], dt_v, isem_t).wait()
        pltpu.make_async_copy(vals_hbm.at[pl.ds(toff, ETAIL)], vt_v, isem_t).wait()
        pltpu.async_copy(emb_hbm.at[st_v], rows_t, gsem_a)
        pltpu.make_async_copy(emb_hbm.at[st_v], rows_t, gsem_a).wait()
        _scale(rows_t, vt_v, ETAIL)
        pltpu.sync_copy(rows_t, acc_sh.at[dt_v], add=True)
        plsc.subcore_barrier()

        out0 = c * NPAD + row0
        for kk in range(ROWS_PER_TILE // ZCHUNK):
            pltpu.sync_copy(acc_sh.at[pl.ds(row0 + kk * ZCHUNK, ZCHUNK)], rows_a)
            pltpu.sync_copy(rows_a, part_hbm.at[pl.ds(out0 + kk * ZCHUNK, ZCHUNK)])

    return k(emb, src, dst, vals)


def _combine(part, acc, last):
    BR = 1280
    bs0 = pl.BlockSpec((BR, D), lambda i: (i, 0))
    bs1 = pl.BlockSpec((BR, D), lambda i: (i + NPAD // BR, 0))

    if last:
        def body(p0_ref, p1_ref, acc_ref, light_ref):
            e = p0_ref[...] + p1_ref[...]
            light_ref[...] = (acc_ref[...] + e) * 0.25
        out_shape = jax.ShapeDtypeStruct((NPAD, D), jnp.float32)
        out_specs = bs0
    else:
        def body(p0_ref, p1_ref, acc_ref, emb_ref, accout_ref):
            e = p0_ref[...] + p1_ref[...]
            emb_ref[...] = e
            accout_ref[...] = acc_ref[...] + e
        out_shape = (jax.ShapeDtypeStruct((NPAD, D), jnp.float32),
                     jax.ShapeDtypeStruct((NPAD, D), jnp.float32))
        out_specs = (bs0, bs0)

    return pl.pallas_call(
        body,
        grid=(NPAD // BR,),
        in_specs=[bs0, bs1, bs0],
        out_specs=out_specs,
        out_shape=out_shape,
    )(part, part, acc)


def _batch_gather(light, users, items, xij, x1, x0):
    mesh = plsc.VectorSubcoreMesh(core_axis_name="c", subcore_axis_name="s")
    BPW = B // NW
    CH = 128
    NCH = BPW // CH

    @functools.partial(
        pl.kernel,
        out_type=(jax.ShapeDtypeStruct((B, D), jnp.float32),
                  jax.ShapeDtypeStruct((B, D), jnp.float32),
                  jax.ShapeDtypeStruct((B,), jnp.float32)),
        mesh=mesh,
        scratch_types=[
            pltpu.VMEM((BPW,), jnp.int32),
            pltpu.VMEM((BPW,), jnp.int32),
            pltpu.VMEM((BPW,), jnp.int32),
            pltpu.VMEM((BPW,), jnp.float32),
            pltpu.VMEM((CH, D), jnp.float32),
            pltpu.VMEM((CH, D), jnp.float32),
            pltpu.VMEM((NUM_ITEMS,), jnp.float32),
            pltpu.VMEM((NUM_ITEMS,), jnp.float32),
            pltpu.SemaphoreType.DMA,
            pltpu.SemaphoreType.DMA,
            pltpu.SemaphoreType.DMA,
            pltpu.SemaphoreType.DMA,
        ],
        compiler_params=pltpu.CompilerParams(needs_layout_passes=False),
    )
    def k(light_hbm, users_hbm, items_hbm, xij_hbm, x1_hbm, x0_hbm,
          urows_hbm, irows_hbm, xsel_hbm,
          uidx_v, iidx_v, xv_v, xsel_v, r0_v, r1_v, x1_v, x0_v,
          gsem0, gsem1, ssem0, ssem1):
        c = lax.axis_index("c")
        s = lax.axis_index("s")
        wid = c * NS + s
        base = wid * BPW
        pltpu.sync_copy(users_hbm.at[pl.ds(base, BPW)], uidx_v)
        pltpu.sync_copy(items_hbm.at[pl.ds(base, BPW)], iidx_v)
        pltpu.sync_copy(xij_hbm.at[pl.ds(base, BPW)], xv_v)
        pltpu.sync_copy(x1_hbm, x1_v)
        pltpu.sync_copy(x0_hbm, x0_v)

        def _g16(g, carry):
            sl = pl.ds(g * 16, 16)
            idx16 = iidx_v[sl]
            v1 = plsc.load_gather(x1_v, [idx16])
            v0 = plsc.load_gather(x0_v, [idx16])
            xsel_v[sl] = jnp.where(xv_v[sl] != 0, v1, v0)
            iidx_v[sl] = idx16 + NUM_USERS
            return carry
        lax.fori_loop(0, BPW // 16, _g16, 0)
        pltpu.async_copy(xsel_v, xsel_hbm.at[pl.ds(base, BPW)], ssem0)
        pltpu.make_async_copy(xsel_v, xsel_hbm.at[pl.ds(base, BPW)], ssem0).wait()

        bufs = (r0_v, r1_v)
        gsems = (gsem0, gsem1)
        ssems = (ssem0, ssem1)

        def _idx(j):
            iv = uidx_v if j < NCH else iidx_v
            return iv.at[pl.ds((j % NCH) * CH, CH)]

        def _dst(j):
            ov = urows_hbm if j < NCH else irows_hbm
            return ov.at[pl.ds(base + (j % NCH) * CH, CH)]

        pltpu.async_copy(light_hbm.at[_idx(0)], bufs[0], gsems[0])
        for j in range(2 * NCH):
            b = j % 2
            pltpu.make_async_copy(light_hbm.at[_idx(j)], bufs[b],
                                  gsems[b]).wait()
            if j + 1 < 2 * NCH:
                if j >= 1:
                    pltpu.make_async_copy(bufs[1 - b], _dst(j - 1),
                                          ssems[1 - b]).wait()
                pltpu.async_copy(light_hbm.at[_idx(j + 1)], bufs[1 - b],
                                 gsems[1 - b])
            pltpu.async_copy(bufs[b], _dst(j), ssems[b])
        pltpu.make_async_copy(bufs[0], _dst(6), ssems[0]).wait()
        pltpu.make_async_copy(bufs[1], _dst(7), ssems[1]).wait()

    return k(light, users, items, xij, x1, x0)


def _score(u_rows, i_rows, xsel2d, wu, wi):
    BT = 2048

    def body(u_ref, i_ref, x_ref, wu_ref, wi_ref, out_ref):
        lu = lax.dot_general(u_ref[...], wu_ref[...], (((1,), (1,)), ((), ())),
                             preferred_element_type=jnp.float32)
        m = jnp.max(lu, axis=1, keepdims=True)
        ex = jnp.exp(lu - m)
        p = ex / jnp.sum(ex, axis=1, keepdims=True)
        li = lax.dot_general(i_ref[...], wi_ref[...], (((1,), (1,)), ((), ())),
                             preferred_element_type=jnp.float32)
        sg = 1.0 / (1.0 + jnp.exp(-li))
        prod = (p * sg).reshape(BT // 128, 128, 128)
        g = 0.5 * jnp.sum(prod, axis=2)
        out_ref[...] = g + 0.5 / (1.0 + jnp.exp(-x_ref[...]))

    return pl.pallas_call(
        body,
        grid=(B // BT,),
        in_specs=[pl.BlockSpec((BT, D), lambda i: (i, 0)),
                  pl.BlockSpec((BT, D), lambda i: (i, 0)),
                  pl.BlockSpec((BT // 128, 128), lambda i: (i, 0)),
                  pl.BlockSpec((D, D), lambda i: (0, 0)),
                  pl.BlockSpec((D, D), lambda i: (0, 0))],
        out_specs=pl.BlockSpec((BT // 128, 128), lambda i: (i, 0)),
        out_shape=jax.ShapeDtypeStruct((B // 128, 128), jnp.float32),
    )(u_rows, i_rows, xsel2d, wu, wi)


def kernel(users, items, xij, edge_index, edge_vals, emb_user, emb_item,
           W_user, W_item, xij_item1, xij_item0):
    all_emb = jnp.concatenate(
        [emb_user, emb_item, jnp.zeros((NPAD - N, D), jnp.float32)], axis=0)
    src = edge_index[0]
    dst = edge_index[1]
    emb = all_emb
    acc = all_emb
    light = None
    for layer in range(3):
        part = _propagate_layer(emb, src, dst, edge_vals)
        if layer < 2:
            emb, acc = _combine(part, acc, last=False)
        else:
            light = _combine(part, acc, last=True)
    u_rows, i_rows, xsel = _batch_gather(
        light, users, items, xij,
        xij_item1.reshape(-1), xij_item0.reshape(-1))
    gamma2d = _score(u_rows, i_rows, xsel.reshape(B // 128, 128),
                     W_user, W_item)
    return gamma2d.reshape(B)


# speedup vs baseline: 11.0433x; 1.4394x over previous
import functools

import jax
import jax.numpy as jnp
from jax import lax
from jax.experimental import pallas as pl
from jax.experimental.pallas import tpu as pltpu
from jax.experimental.pallas import tpu_sc as plsc

NUM_USERS = 5000
NUM_ITEMS = 5000
N = NUM_USERS + NUM_ITEMS
D = 128
E = 320000
B = 16384
NC = 2
NS = 16
NW = NC * NS
EPW = E // NW
ECHUNK = 80
ENCHUNKS = EPW // ECHUNK
NPAD = 10240
ROWS_PER_TILE = NPAD // NS
ZCHUNK = ECHUNK


def _propagate_layer(emb, src, dst, vals):
    mesh = plsc.VectorSubcoreMesh(core_axis_name="c", subcore_axis_name="s")
    NBUF = 4
    NSET = 8
    PER = 15

    @functools.partial(
        pl.kernel,
        out_type=jax.ShapeDtypeStruct((2 * NPAD, D), jnp.float32),
        mesh=mesh,
        scratch_types=(
            [pltpu.VMEM((ECHUNK,), jnp.int32) for _ in range(NSET)]
            + [pltpu.VMEM((ECHUNK,), jnp.int32) for _ in range(NSET)]
            + [pltpu.VMEM((ECHUNK,), jnp.float32) for _ in range(NSET)]
            + [pltpu.VMEM((ECHUNK, D), jnp.float32) for _ in range(NBUF)]
            + [pltpu.VMEM_SHARED((NPAD, D), jnp.float32)]
            + [pltpu.SemaphoreType.DMA for _ in range(NSET)]
            + [pltpu.SemaphoreType.DMA for _ in range(NBUF)]
            + [pltpu.SemaphoreType.DMA for _ in range(NBUF)]
        ),
        compiler_params=pltpu.CompilerParams(needs_layout_passes=False),
    )
    def k(emb_hbm, src_hbm, dst_hbm, vals_hbm, part_hbm, *scr):
        sv = scr[0:NSET]
        dv = scr[NSET:2 * NSET]
        vv = scr[2 * NSET:3 * NSET]
        bufs = scr[3 * NSET:3 * NSET + NBUF]
        acc_sh = scr[3 * NSET + NBUF]
        isem = scr[3 * NSET + NBUF + 1:3 * NSET + NBUF + 1 + NSET]
        gsem = scr[3 * NSET + NBUF + 1 + NSET:3 * NSET + NBUF + 1 + NSET + NBUF]
        ssem = scr[3 * NSET + NBUF + 1 + NSET + NBUF:]
        c = lax.axis_index("c")
        s = lax.axis_index("s")
        wid = c * NS + s
        ebase = wid * EPW

        def _idx_start(i, st):
            off = ebase + i * ECHUNK
            pltpu.async_copy(src_hbm.at[pl.ds(off, ECHUNK)], sv[st], isem[st])
            pltpu.async_copy(dst_hbm.at[pl.ds(off, ECHUNK)], dv[st], isem[st])
            pltpu.async_copy(vals_hbm.at[pl.ds(off, ECHUNK)], vv[st], isem[st])

        def _idx_wait(i, st):
            off = ebase + i * ECHUNK
            pltpu.make_async_copy(src_hbm.at[pl.ds(off, ECHUNK)], sv[st],
                                  isem[st]).wait()
            pltpu.make_async_copy(dst_hbm.at[pl.ds(off, ECHUNK)], dv[st],
                                  isem[st]).wait()
            pltpu.make_async_copy(vals_hbm.at[pl.ds(off, ECHUNK)], vv[st],
                                  isem[st]).wait()

        def _scale(b, st):
            def _e(e, c2):
                sp = plsc.load_gather(vv[st], [jnp.full((16,), e, jnp.int32)])
                for j in range(D // 16):
                    sl = pl.ds(j * 16, 16)
                    bufs[b][e, sl] = bufs[b][e, sl] * sp
                return c2
            lax.fori_loop(0, ECHUNK, _e, 0, unroll=2)

        def _zrow(r, carry):
            for j in range(D // 16):
                bufs[0][r, pl.ds(j * 16, 16)] = jnp.zeros((16,), jnp.float32)
            return carry
        lax.fori_loop(0, ZCHUNK, _zrow, 0)
        row0 = s * ROWS_PER_TILE
        for kk in range(ROWS_PER_TILE // ZCHUNK):
            pltpu.sync_copy(bufs[0], acc_sh.at[pl.ds(row0 + kk * ZCHUNK,
                                                     ZCHUNK)])

        for i in range(5):
            _idx_start(i, i)
        for i in range(3):
            _idx_wait(i, i)
            pltpu.async_copy(emb_hbm.at[sv[i]], bufs[i], gsem[i])
        plsc.subcore_barrier()

        for ks in range(8):
            i = ks
            b = ks % NBUF
            st = ks % NSET
            gb = (ks + 3) % NBUF
            gst = (ks + 3) % NSET
            _idx_wait(i + 3, gst)
            pltpu.make_async_copy(emb_hbm.at[sv[st]], bufs[b],
                                  gsem[b]).wait()
            _scale(b, st)
            if ks >= 1:
                pltpu.make_async_copy(bufs[gb], acc_sh.at[dv[gst]],
                                      ssem[gb]).wait()
            pltpu.async_copy(emb_hbm.at[sv[gst]], bufs[gb], gsem[gb])
            _idx_start(i + 5, (ks + 5) % NSET)
            pltpu.async_copy(bufs[b], acc_sh.at[dv[st]], ssem[b], add=True)

        def _iter(q, carry):
            i0 = 8 * q + 8
            for ks in range(8):
                i = i0 + ks
                b = ks % NBUF
                st = ks % NSET
                gb = (ks + 3) % NBUF
                gst = (ks + 3) % NSET
                _idx_wait(i + 3, gst)
                pltpu.make_async_copy(emb_hbm.at[sv[st]], bufs[b],
                                      gsem[b]).wait()
                _scale(b, st)
                pltpu.make_async_copy(bufs[gb], acc_sh.at[dv[gst]],
                                      ssem[gb]).wait()
                pltpu.async_copy(emb_hbm.at[sv[gst]], bufs[gb], gsem[gb])
                _idx_start(i + 5, (ks + 5) % NSET)
                pltpu.async_copy(bufs[b], acc_sh.at[dv[st]], ssem[b],
                                 add=True)
            return carry
        lax.fori_loop(0, PER - 1, _iter, 0)

        for ks in range(5):
            i = 120 + ks
            b = i % NBUF
            st = i % NSET
            gb = (i + 3) % NBUF
            gst = (i + 3) % NSET
            if i + 3 <= ENCHUNKS - 1:
                _idx_wait(i + 3, gst)
            pltpu.make_async_copy(emb_hbm.at[sv[st]], bufs[b],
                                  gsem[b]).wait()
            _scale(b, st)
            if i + 3 <= ENCHUNKS - 1:
                pltpu.make_async_copy(bufs[gb], acc_sh.at[dv[gst]],
                                      ssem[gb]).wait()
                pltpu.async_copy(emb_hbm.at[sv[gst]], bufs[gb], gsem[gb])
            pltpu.async_copy(bufs[b], acc_sh.at[dv[st]], ssem[b], add=True)
        for i in range(121, 125):
            b = i % NBUF
            st = i % NSET
            pltpu.make_async_copy(bufs[b], acc_sh.at[dv[st]], ssem[b]).wait()
        plsc.subcore_barrier()

        out0 = c * NPAD + row0
        for kk in range(ROWS_PER_TILE // ZCHUNK):
            pltpu.sync_copy(acc_sh.at[pl.ds(row0 + kk * ZCHUNK, ZCHUNK)],
                            bufs[0])
            pltpu.sync_copy(bufs[0], part_hbm.at[pl.ds(out0 + kk * ZCHUNK,
                                                       ZCHUNK)])

    return k(emb, src, dst, vals)


def _combine(part, acc, last):
    BR = 1280
    bs0 = pl.BlockSpec((BR, D), lambda i: (i, 0))
    bs1 = pl.BlockSpec((BR, D), lambda i: (i + NPAD // BR, 0))

    if last:
        def body(p0_ref, p1_ref, acc_ref, light_ref):
            e = p0_ref[...] + p1_ref[...]
            light_ref[...] = (acc_ref[...] + e) * 0.25
        out_shape = jax.ShapeDtypeStruct((NPAD, D), jnp.float32)
        out_specs = bs0
    else:
        def body(p0_ref, p1_ref, acc_ref, emb_ref, accout_ref):
            e = p0_ref[...] + p1_ref[...]
            emb_ref[...] = e
            accout_ref[...] = acc_ref[...] + e
        out_shape = (jax.ShapeDtypeStruct((NPAD, D), jnp.float32),
                     jax.ShapeDtypeStruct((NPAD, D), jnp.float32))
        out_specs = (bs0, bs0)

    return pl.pallas_call(
        body,
        grid=(NPAD // BR,),
        in_specs=[bs0, bs1, bs0],
        out_specs=out_specs,
        out_shape=out_shape,
    )(part, part, acc)


def _batch_gather(light, users, items, xij, x1, x0):
    mesh = plsc.VectorSubcoreMesh(core_axis_name="c", subcore_axis_name="s")
    BPW = B // NW
    CH = 128
    NCH = BPW // CH

    @functools.partial(
        pl.kernel,
        out_type=(jax.ShapeDtypeStruct((B, D), jnp.float32),
                  jax.ShapeDtypeStruct((B, D), jnp.float32),
                  jax.ShapeDtypeStruct((B,), jnp.float32)),
        mesh=mesh,
        scratch_types=[
            pltpu.VMEM((BPW,), jnp.int32),
            pltpu.VMEM((BPW,), jnp.int32),
            pltpu.VMEM((BPW,), jnp.int32),
            pltpu.VMEM((BPW,), jnp.float32),
            pltpu.VMEM((CH, D), jnp.float32),
            pltpu.VMEM((CH, D), jnp.float32),
            pltpu.VMEM((NUM_ITEMS,), jnp.float32),
            pltpu.VMEM((NUM_ITEMS,), jnp.float32),
            pltpu.SemaphoreType.DMA,
            pltpu.SemaphoreType.DMA,
            pltpu.SemaphoreType.DMA,
            pltpu.SemaphoreType.DMA,
        ],
        compiler_params=pltpu.CompilerParams(needs_layout_passes=False),
    )
    def k(light_hbm, users_hbm, items_hbm, xij_hbm, x1_hbm, x0_hbm,
          urows_hbm, irows_hbm, xsel_hbm,
          uidx_v, iidx_v, xv_v, xsel_v, r0_v, r1_v, x1_v, x0_v,
          gsem0, gsem1, ssem0, ssem1):
        c = lax.axis_index("c")
        s = lax.axis_index("s")
        wid = c * NS + s
        base = wid * BPW
        pltpu.sync_copy(users_hbm.at[pl.ds(base, BPW)], uidx_v)
        pltpu.sync_copy(items_hbm.at[pl.ds(base, BPW)], iidx_v)
        pltpu.sync_copy(xij_hbm.at[pl.ds(base, BPW)], xv_v)
        pltpu.sync_copy(x1_hbm, x1_v)
        pltpu.sync_copy(x0_hbm, x0_v)

        def _g16(g, carry):
            sl = pl.ds(g * 16, 16)
            idx16 = iidx_v[sl]
            v1 = plsc.load_gather(x1_v, [idx16])
            v0 = plsc.load_gather(x0_v, [idx16])
            xsel_v[sl] = jnp.where(xv_v[sl] != 0, v1, v0)
            iidx_v[sl] = idx16 + NUM_USERS
            return carry
        lax.fori_loop(0, BPW // 16, _g16, 0)
        pltpu.async_copy(xsel_v, xsel_hbm.at[pl.ds(base, BPW)], ssem0)
        pltpu.make_async_copy(xsel_v, xsel_hbm.at[pl.ds(base, BPW)], ssem0).wait()

        bufs = (r0_v, r1_v)
        gsems = (gsem0, gsem1)
        ssems = (ssem0, ssem1)

        def _idx(j):
            iv = uidx_v if j < NCH else iidx_v
            return iv.at[pl.ds((j % NCH) * CH, CH)]

        def _dst(j):
            ov = urows_hbm if j < NCH else irows_hbm
            return ov.at[pl.ds(base + (j % NCH) * CH, CH)]

        pltpu.async_copy(light_hbm.at[_idx(0)], bufs[0], gsems[0])
        for j in range(2 * NCH):
            b = j % 2
            pltpu.make_async_copy(light_hbm.at[_idx(j)], bufs[b],
                                  gsems[b]).wait()
            if j + 1 < 2 * NCH:
                if j >= 1:
                    pltpu.make_async_copy(bufs[1 - b], _dst(j - 1),
                                          ssems[1 - b]).wait()
                pltpu.async_copy(light_hbm.at[_idx(j + 1)], bufs[1 - b],
                                 gsems[1 - b])
            pltpu.async_copy(bufs[b], _dst(j), ssems[b])
        pltpu.make_async_copy(bufs[0], _dst(6), ssems[0]).wait()
        pltpu.make_async_copy(bufs[1], _dst(7), ssems[1]).wait()

    return k(light, users, items, xij, x1, x0)


def _score(u_rows, i_rows, xsel2d, wu, wi):
    BT = 2048

    def body(u_ref, i_ref, x_ref, wu_ref, wi_ref, out_ref):
        lu = lax.dot_general(u_ref[...], wu_ref[...], (((1,), (1,)), ((), ())),
                             preferred_element_type=jnp.float32)
        m = jnp.max(lu, axis=1, keepdims=True)
        ex = jnp.exp(lu - m)
        p = ex / jnp.sum(ex, axis=1, keepdims=True)
        li = lax.dot_general(i_ref[...], wi_ref[...], (((1,), (1,)), ((), ())),
                             preferred_element_type=jnp.float32)
        sg = 1.0 / (1.0 + jnp.exp(-li))
        prod = (p * sg).reshape(BT // 128, 128, 128)
        g = 0.5 * jnp.sum(prod, axis=2)
        out_ref[...] = g + 0.5 / (1.0 + jnp.exp(-x_ref[...]))

    return pl.pallas_call(
        body,
        grid=(B // BT,),
        in_specs=[pl.BlockSpec((BT, D), lambda i: (i, 0)),
                  pl.BlockSpec((BT, D), lambda i: (i, 0)),
                  pl.BlockSpec((BT // 128, 128), lambda i: (i, 0)),
                  pl.BlockSpec((D, D), lambda i: (0, 0)),
                  pl.BlockSpec((D, D), lambda i: (0, 0))],
        out_specs=pl.BlockSpec((BT // 128, 128), lambda i: (i, 0)),
        out_shape=jax.ShapeDtypeStruct((B // 128, 128), jnp.float32),
    )(u_rows, i_rows, xsel2d, wu, wi)


def kernel(users, items, xij, edge_index, edge_vals, emb_user, emb_item,
           W_user, W_item, xij_item1, xij_item0):
    all_emb = jnp.concatenate(
        [emb_user, emb_item, jnp.zeros((NPAD - N, D), jnp.float32)], axis=0)
    src = edge_index[0]
    dst = edge_index[1]
    emb = all_emb
    acc = all_emb
    light = None
    for layer in range(3):
        part = _propagate_layer(emb, src, dst, edge_vals)
        if layer < 2:
            emb, acc = _combine(part, acc, last=False)
        else:
            light = _combine(part, acc, last=True)
    u_rows, i_rows, xsel = _batch_gather(
        light, users, items, xij,
        xij_item1.reshape(-1), xij_item0.reshape(-1))
    gamma2d = _score(u_rows, i_rows, xsel.reshape(B // 128, 128),
                     W_user, W_item)
    return gamma2d.reshape(B)
